# Initial kernel scaffold; baseline (speedup 1.0000x reference)
import jax
import jax.numpy as jnp
from jax.experimental import pallas as pl


def kernel(x, table0, tables, W1, b1, W2, b2, W3, b3):
    raise NotImplementedError("write your pallas kernel here")


# speedup vs baseline: 6.0394x; 6.0394x over previous
import functools

import jax
import jax.numpy as jnp
from jax import lax
from jax.experimental import pallas as pl
from jax.experimental.pallas import tpu as pltpu
from jax.experimental.pallas import tpu_sc as plsc

B = 16384
HIDDEN = 256
ACTIONS = 64
EMB = 16
NFIELDS = 17
CONCAT = NFIELDS * EMB
TABLE_ROWS = 25 + (NFIELDS - 1) * 10

NC, NS = 2, 16
NW = NC * NS
R = B * NFIELDS
R_PER_W = R // NW
IDX_MINOR = 128
STREAMS_PER_CHUNK = 17
CHUNK = STREAMS_PER_CHUNK * IDX_MINOR
NCHUNK = R_PER_W // CHUNK


IDX_ROWS_PER_W = R_PER_W // IDX_MINOR


def _sc_gather(flat_table, gidx3d):
    mesh = plsc.VectorSubcoreMesh(core_axis_name="c", subcore_axis_name="s")

    @functools.partial(
        pl.kernel,
        mesh=mesh,
        out_type=jax.ShapeDtypeStruct((R, EMB), jnp.float32),
        scratch_types=[
            pltpu.VMEM((IDX_ROWS_PER_W, IDX_MINOR), jnp.int32),
            pltpu.VMEM((CHUNK, EMB), jnp.float32),
            pltpu.SemaphoreType.DMA,
        ],
        compiler_params=pltpu.CompilerParams(use_tc_tiling_on_sc=False),
    )
    def gather_kernel(tbl_hbm, gidx_hbm, out_hbm, idx_v, rows_v, sem):
        wid = lax.axis_index("s") * NC + lax.axis_index("c")
        base = wid * R_PER_W

        pltpu.sync_copy(gidx_hbm.at[wid], idx_v)

        def chunk_body(k, carry):
            handles = [
                pltpu.async_copy(
                    tbl_hbm.at[idx_v.at[k * STREAMS_PER_CHUNK + j]],
                    rows_v.at[pl.ds(j * IDX_MINOR, IDX_MINOR)],
                    sem,
                )
                for j in range(STREAMS_PER_CHUNK)
            ]
            for h in handles:
                h.wait()
            pltpu.sync_copy(rows_v, out_hbm.at[pl.ds(base + k * CHUNK, CHUNK)])
            return carry

        lax.fori_loop(0, NCHUNK, chunk_body, 0)

    return gather_kernel(flat_table, gidx3d)


def _mlp(emb, W1, b1, W2, b2, W3, b3):
    BB = 1024

    def body(emb_ref, w1_ref, b1_ref, w2_ref, b2_ref, w3_ref, b3_ref, out_ref):
        h = emb_ref[...]
        h = jnp.maximum(
            jnp.dot(h, w1_ref[...], preferred_element_type=jnp.float32)
            + b1_ref[...],
            0.0,
        )
        h = jnp.maximum(
            jnp.dot(h, w2_ref[...], preferred_element_type=jnp.float32)
            + b2_ref[...],
            0.0,
        )
        logits = (
            jnp.dot(h, w3_ref[...], preferred_element_type=jnp.float32)
            + b3_ref[...]
        )
        m = jnp.max(logits, axis=1, keepdims=True)
        e = jnp.exp(logits - m)
        out_ref[...] = e / jnp.sum(e, axis=1, keepdims=True)

    return pl.pallas_call(
        body,
        grid=(B // BB,),
        in_specs=[
            pl.BlockSpec((BB, CONCAT), lambda i: (i, 0)),
            pl.BlockSpec((CONCAT, HIDDEN), lambda i: (0, 0)),
            pl.BlockSpec((1, HIDDEN), lambda i: (0, 0)),
            pl.BlockSpec((HIDDEN, HIDDEN), lambda i: (0, 0)),
            pl.BlockSpec((1, HIDDEN), lambda i: (0, 0)),
            pl.BlockSpec((HIDDEN, ACTIONS), lambda i: (0, 0)),
            pl.BlockSpec((1, ACTIONS), lambda i: (0, 0)),
        ],
        out_specs=pl.BlockSpec((BB, ACTIONS), lambda i: (i, 0)),
        out_shape=jax.ShapeDtypeStruct((B, ACTIONS), jnp.float32),
    )(emb, W1, b1, W2, b2, W3, b3)


def kernel(x, table0, tables, W1, b1, W2, b2, W3, b3):
    flat_table = jnp.concatenate([table0, tables.reshape(-1, EMB)], axis=0)
    offs = jnp.concatenate(
        [
            jnp.zeros((1,), jnp.int32),
            25 + 10 * jnp.arange(NFIELDS - 1, dtype=jnp.int32),
        ]
    )
    gidx = (x.astype(jnp.int32) + offs[None, :]).reshape(
        NW, IDX_ROWS_PER_W, IDX_MINOR
    )
    emb_rows = _sc_gather(flat_table, gidx)
    emb = emb_rows.reshape(B, CONCAT)
    return _mlp(
        emb,
        W1,
        b1.reshape(1, HIDDEN),
        W2,
        b2.reshape(1, HIDDEN),
        W3,
        b3.reshape(1, ACTIONS),
    )


# speedup vs baseline: 9.4943x; 1.5721x over previous
import functools

import jax
import jax.numpy as jnp
from jax import lax
from jax.experimental import pallas as pl
from jax.experimental.pallas import tpu as pltpu
from jax.experimental.pallas import tpu_sc as plsc

B = 16384
HIDDEN = 256
ACTIONS = 64
EMB = 16
NFIELDS = 17
CONCAT = NFIELDS * EMB
TABLE_ROWS = 25 + (NFIELDS - 1) * 10

NC, NS = 2, 16
NW = NC * NS
R = B * NFIELDS
R_PER_W = R // NW
IDX_MINOR = 128
STREAMS_PER_CHUNK = 17
CHUNK = STREAMS_PER_CHUNK * IDX_MINOR
NCHUNK = R_PER_W // CHUNK


IDX_ROWS_PER_W = R_PER_W // IDX_MINOR
B_PER_CHUNK = CHUNK // NFIELDS
B_PER_W = B // NW


def _sc_gather(flat_table, gidx3d):
    mesh = plsc.VectorSubcoreMesh(core_axis_name="c", subcore_axis_name="s")

    @functools.partial(
        pl.kernel,
        mesh=mesh,
        out_type=jax.ShapeDtypeStruct((R, EMB), jnp.float32),
        scratch_types=[
            pltpu.VMEM((TABLE_ROWS, EMB), jnp.float32),
            pltpu.VMEM_SHARED((TABLE_ROWS, EMB), jnp.float32),
            pltpu.VMEM((IDX_ROWS_PER_W, IDX_MINOR), jnp.int32),
            pltpu.VMEM((2, CHUNK, EMB), jnp.float32),
            pltpu.SemaphoreType.DMA,
        ],
        compiler_params=pltpu.CompilerParams(use_tc_tiling_on_sc=False),
    )
    def gather_kernel(tbl_hbm, gidx_hbm, out_hbm, tbl_v, tbl_s, idx_v, rows_v, sem):
        sid = lax.axis_index("s")
        wid = sid * NC + lax.axis_index("c")
        base_b = wid * B_PER_W

        @pl.when(sid == 0)
        def _stage_table():
            pltpu.sync_copy(tbl_hbm, tbl_v)
            pltpu.sync_copy(tbl_v, tbl_s)

        plsc.subcore_barrier()
        pltpu.sync_copy(gidx_hbm.at[wid], idx_v)

        def fire(k, p):
            return [
                pltpu.async_copy(
                    tbl_s.at[idx_v.at[k * STREAMS_PER_CHUNK + j]],
                    rows_v.at[p].at[pl.ds(j * IDX_MINOR, IDX_MINOR)],
                    sem,
                )
                for j in range(STREAMS_PER_CHUNK)
            ]

        handles = fire(0, 0)
        for k in range(NCHUNK):
            nxt = fire(k + 1, (k + 1) % 2) if k + 1 < NCHUNK else []
            for h in handles:
                h.wait()
            handles = nxt
            pltpu.sync_copy(
                rows_v.at[k % 2],
                out_hbm.at[pl.ds((base_b + k * B_PER_CHUNK) * NFIELDS, CHUNK)],
            )

    return gather_kernel(flat_table, gidx3d)


def _mlp(emb, W1, b1, W2, b2, W3, b3):
    BB = 1024

    def body(emb_ref, w1_ref, b1_ref, w2_ref, b2_ref, w3_ref, b3_ref, out_ref):
        h = emb_ref[...]
        h = jnp.maximum(
            jnp.dot(h, w1_ref[...], preferred_element_type=jnp.float32)
            + b1_ref[...],
            0.0,
        )
        h = jnp.maximum(
            jnp.dot(h, w2_ref[...], preferred_element_type=jnp.float32)
            + b2_ref[...],
            0.0,
        )
        logits = (
            jnp.dot(h, w3_ref[...], preferred_element_type=jnp.float32)
            + b3_ref[...]
        )
        m = jnp.max(logits, axis=1, keepdims=True)
        e = jnp.exp(logits - m)
        out_ref[...] = e / jnp.sum(e, axis=1, keepdims=True)

    return pl.pallas_call(
        body,
        grid=(B // BB,),
        in_specs=[
            pl.BlockSpec((BB, CONCAT), lambda i: (i, 0)),
            pl.BlockSpec((CONCAT, HIDDEN), lambda i: (0, 0)),
            pl.BlockSpec((1, HIDDEN), lambda i: (0, 0)),
            pl.BlockSpec((HIDDEN, HIDDEN), lambda i: (0, 0)),
            pl.BlockSpec((1, HIDDEN), lambda i: (0, 0)),
            pl.BlockSpec((HIDDEN, ACTIONS), lambda i: (0, 0)),
            pl.BlockSpec((1, ACTIONS), lambda i: (0, 0)),
        ],
        out_specs=pl.BlockSpec((BB, ACTIONS), lambda i: (i, 0)),
        out_shape=jax.ShapeDtypeStruct((B, ACTIONS), jnp.float32),
    )(emb, W1, b1, W2, b2, W3, b3)


def kernel(x, table0, tables, W1, b1, W2, b2, W3, b3):
    flat_table = jnp.concatenate([table0, tables.reshape(-1, EMB)], axis=0)
    offs = jnp.concatenate(
        [
            jnp.zeros((1,), jnp.int32),
            25 + 10 * jnp.arange(NFIELDS - 1, dtype=jnp.int32),
        ]
    )
    gidx = (x.astype(jnp.int32) + offs[None, :]).reshape(
        NW, IDX_ROWS_PER_W, IDX_MINOR
    )
    emb = _sc_gather(flat_table, gidx).reshape(B, CONCAT)
    return _mlp(
        emb,
        W1,
        b1.reshape(1, HIDDEN),
        W2,
        b2.reshape(1, HIDDEN),
        W3,
        b3.reshape(1, ACTIONS),
    )
